# baseline (speedup 1.0000x reference)
import functools

import jax
import jax.numpy as jnp
from jax import lax
from jax.experimental import pallas as pl
from jax.experimental.pallas import tpu as pltpu
from jax.experimental.pallas import tpu_sc as plsc

N_NODES = 100000
N_EDGES = 3200000

LANE = 128
ROWS = N_EDGES // LANE
NPAD = 100352
TROWS = NPAD // LANE
HALF = NPAD // 2
HROWS = 400
UROWS = NPAD // 256

D_RPT = 776
D_TAIL = ROWS - D_RPT * 32
A_RPT = 1560
A_TAIL = ROWS - A_RPT * 16

_mesh = plsc.VectorSubcoreMesh(core_axis_name="c", subcore_axis_name="s")
_cp = pltpu.CompilerParams(use_tc_tiling_on_sc=False, needs_layout_passes=False)

_f32 = jnp.float32
_i32 = jnp.int32


def _iota16():
    return lax.iota(_i32, 16)


def _zero2d(acc, nrows):
    def z(i, _):
        r16 = jnp.broadcast_to(lax.shift_right_logical(i, 3), (16,))
        c16 = lax.bitwise_and(i, 7) * 16 + _iota16()
        plsc.store_scatter(acc, [r16, c16], jnp.zeros((16,), _f32))
        return 0
    lax.fori_loop(0, nrows * 8, z, 0)


@functools.partial(
    pl.kernel,
    out_type=jax.ShapeDtypeStruct((32, TROWS, LANE), _f32),
    mesh=_mesh, compiler_params=_cp,
    scratch_types=[
        pltpu.VMEM((TROWS, LANE), _f32),
        pltpu.VMEM((24, LANE), _i32),
    ],
)
def _deg_pass(ei, out, acc, dstb):
    c = lax.axis_index("c")
    s = lax.axis_index("s")
    w = c * 16 + s
    _zero2d(acc, TROWS)

    ones = jnp.ones((16,), _f32)

    def do_rows(nr):
        for r in range(nr):
            for g in range(8):
                dst16 = dstb[r, pl.ds(g * 16, 16)]
                plsc.addupdate_scatter(
                    acc, [lax.shift_right_logical(dst16, 7),
                          lax.bitwise_and(dst16, 127)], ones)

    def chunk(k, _):
        pltpu.sync_copy(ei.at[1, pl.ds(w * D_RPT + k * 24, 24)], dstb)
        do_rows(24)
        return 0
    lax.fori_loop(0, 32, chunk, 0)
    pltpu.sync_copy(ei.at[1, pl.ds(w * D_RPT + 768, 8)], dstb.at[pl.ds(0, 8)])
    do_rows(8)

    @pl.when(w < D_TAIL // 8)
    def _():
        pltpu.sync_copy(ei.at[1, pl.ds(32 * D_RPT + w * 8, 8)],
                        dstb.at[pl.ds(0, 8)])
        do_rows(8)

    pltpu.sync_copy(acc, out.at[w])


@functools.partial(
    pl.kernel,
    out_type=jax.ShapeDtypeStruct((32, HROWS, LANE), _f32),
    mesh=_mesh, compiler_params=_cp,
    scratch_types=[
        pltpu.VMEM((UROWS, LANE), _i32),
        pltpu.VMEM((HROWS, LANE), _f32),
        pltpu.VMEM((30, LANE), _i32),
        pltpu.VMEM((30, LANE), _i32),
    ],
)
def _agg_pass(ei, uw, out, utab, acc, srcb, dstb):
    c = lax.axis_index("c")
    s = lax.axis_index("s")
    w = c * 16 + s
    base = c * HALF

    pltpu.sync_copy(uw, utab)
    _zero2d(acc, HROWS)

    def do_rows(nr):
        for r in range(nr):
            for g in range(8):
                sl = pl.ds(g * 16, 16)
                src16 = srcb[r, sl]
                dst16 = dstb[r, sl]
                local = dst16 - base
                m = lax.bitwise_and(local >= 0, local < HALF)
                word = lax.shift_right_logical(src16, 1)
                w16 = plsc.load_gather(
                    utab, [lax.shift_right_logical(word, 7),
                           lax.bitwise_and(word, 127)], mask=m)
                sh = lax.shift_left(lax.bitwise_and(src16, 1), 4)
                v = plsc.bitcast(
                    lax.shift_left(lax.shift_right_logical(w16, sh), 16), _f32)
                idxc = jnp.where(m, local, 0)
                plsc.addupdate_scatter(
                    acc, [lax.shift_right_logical(idxc, 7),
                          lax.bitwise_and(idxc, 127)], v, mask=m)

    def chunk(k, _):
        row = s * A_RPT + k * 30
        pltpu.sync_copy(ei.at[0, pl.ds(row, 30)], srcb)
        pltpu.sync_copy(ei.at[1, pl.ds(row, 30)], dstb)
        do_rows(30)
        return 0
    lax.fori_loop(0, A_RPT // 30, chunk, 0)

    @pl.when(s < A_TAIL // 8)
    def _():
        row = 16 * A_RPT + s * 8
        pltpu.sync_copy(ei.at[0, pl.ds(row, 8)], srcb.at[pl.ds(0, 8)])
        pltpu.sync_copy(ei.at[1, pl.ds(row, 8)], dstb.at[pl.ds(0, 8)])
        do_rows(8)

    pltpu.sync_copy(acc, out.at[w])


def _prep_body(degp, x, dis_o, u1_o):
    deg = jnp.sum(degp[...], axis=0) + 1.0
    dis = lax.rsqrt(deg)
    dis_o[...] = dis
    u1_o[...] = dis * x[...]


def _gfull(g_ref):
    g = jnp.sum(g_ref[...], axis=1)
    return jnp.concatenate([g[0, :TROWS // 2], g[1, :TROWS // 2]], axis=0)


def _mid_body(g1, u1, dis, w1, b1, w2, u2_o):
    s1 = dis[...] * (_gfull(g1) + u1[...])
    acc = jnp.zeros_like(s1)
    for j in range(16):
        acc += jnp.maximum(s1 * w1[0, j] + b1[j], 0.0) * w2[j, 0]
    u2_o[...] = dis[...] * acc


def _fin_body(g2, u2, dis, b2, out_o):
    out_o[...] = dis[...] * (_gfull(g2) + u2[...]) + b2[0]


_vec2 = jax.ShapeDtypeStruct((TROWS, LANE), _f32)
_smem = pl.BlockSpec(memory_space=pltpu.SMEM)

_prep_call = pl.pallas_call(_prep_body, out_shape=(_vec2, _vec2))
_mid_call = pl.pallas_call(
    _mid_body,
    in_specs=[pl.BlockSpec(), pl.BlockSpec(), pl.BlockSpec(), _smem, _smem, _smem],
    out_shape=_vec2,
)
_fin_call = pl.pallas_call(
    _fin_body,
    in_specs=[pl.BlockSpec(), pl.BlockSpec(), pl.BlockSpec(), _smem],
    out_shape=_vec2,
)


def _pack_words(u2d):
    b16 = jax.lax.bitcast_convert_type(
        u2d.astype(jnp.bfloat16), jnp.uint16).astype(jnp.uint32)
    flat = b16.reshape(-1)
    words = flat[0::2] | (flat[1::2] << 16)
    return words.astype(_i32).reshape(UROWS, LANE)


def kernel(x, edge_index, W1, b1, W2, b2):
    n = x.shape[0]
    ei = edge_index.astype(_i32).reshape(2, ROWS, LANE)
    xp = jnp.pad(x[:, 0], (0, NPAD - n)).reshape(TROWS, LANE)

    degp = _deg_pass(ei)
    dis, u1 = _prep_call(degp, xp)
    g1 = _agg_pass(ei, _pack_words(u1)).reshape(2, 16, HROWS, LANE)
    u2 = _mid_call(g1, u1, dis, W1, b1, W2)
    g2 = _agg_pass(ei, _pack_words(u2)).reshape(2, 16, HROWS, LANE)
    out = _fin_call(g2, u2, dis, b2)
    return out.reshape(NPAD)[:n].reshape(n, 1)


# speedup vs baseline: 1.2813x; 1.2813x over previous
import functools

import jax
import jax.numpy as jnp
from jax import lax
from jax.experimental import pallas as pl
from jax.experimental.pallas import tpu as pltpu
from jax.experimental.pallas import tpu_sc as plsc

N_NODES = 100000
N_EDGES = 3200000

LANE = 128
ROWS = N_EDGES // LANE
NPAD = 100352
TROWS = NPAD // LANE
HALF = NPAD // 2
HROWS = 400
UROWS = NPAD // 256

D_RPT = 776
D_TAIL = ROWS - D_RPT * 32
A_RPT = 1560
A_TAIL = ROWS - A_RPT * 16

_mesh = plsc.VectorSubcoreMesh(core_axis_name="c", subcore_axis_name="s")
_cp = pltpu.CompilerParams(use_tc_tiling_on_sc=False, needs_layout_passes=False)

_f32 = jnp.float32
_i32 = jnp.int32


def _iota16():
    return lax.iota(_i32, 16)


def _zero2d(acc, nrows):
    def z(i, _):
        r16 = jnp.broadcast_to(lax.shift_right_logical(i, 3), (16,))
        c16 = lax.bitwise_and(i, 7) * 16 + _iota16()
        plsc.store_scatter(acc, [r16, c16], jnp.zeros((16,), _f32))
        return 0
    lax.fori_loop(0, nrows * 8, z, 0)


@functools.partial(
    pl.kernel,
    out_type=jax.ShapeDtypeStruct((32, TROWS, LANE), _f32),
    mesh=_mesh, compiler_params=_cp,
    scratch_types=[
        pltpu.VMEM((TROWS, LANE), _f32),
        pltpu.VMEM((24, LANE), _i32),
    ],
)
def _deg_pass(ei, out, acc, dstb):
    c = lax.axis_index("c")
    s = lax.axis_index("s")
    w = c * 16 + s
    _zero2d(acc, TROWS)

    ones = jnp.ones((16,), _f32)

    def do_rows(nr):
        for r in range(nr):
            for g in range(8):
                dst16 = dstb[r, pl.ds(g * 16, 16)]
                plsc.addupdate_scatter(
                    acc, [lax.shift_right_logical(dst16, 7),
                          lax.bitwise_and(dst16, 127)], ones)

    def chunk(k, _):
        pltpu.sync_copy(ei.at[1, pl.ds(w * D_RPT + k * 24, 24)], dstb)
        do_rows(24)
        return 0
    lax.fori_loop(0, 32, chunk, 0)
    pltpu.sync_copy(ei.at[1, pl.ds(w * D_RPT + 768, 8)], dstb.at[pl.ds(0, 8)])
    do_rows(8)

    @pl.when(w < D_TAIL // 8)
    def _():
        pltpu.sync_copy(ei.at[1, pl.ds(32 * D_RPT + w * 8, 8)],
                        dstb.at[pl.ds(0, 8)])
        do_rows(8)

    pltpu.sync_copy(acc, out.at[w])


@functools.partial(
    pl.kernel,
    out_type=jax.ShapeDtypeStruct((32, HROWS, LANE), _f32),
    mesh=_mesh, compiler_params=_cp,
    scratch_types=[
        pltpu.VMEM((UROWS, LANE), _i32),
        pltpu.VMEM((HROWS, LANE), _f32),
        pltpu.VMEM((30, LANE), _i32),
        pltpu.VMEM((30, LANE), _i32),
        pltpu.VMEM((30, LANE), _i32),
        pltpu.VMEM((30, LANE), _i32),
        pltpu.SemaphoreType.DMA,
        pltpu.SemaphoreType.DMA,
    ],
)
def _agg_pass(ei, uw, out, utab, acc, srcb0, dstb0, srcb1, dstb1, sem0, sem1):
    c = lax.axis_index("c")
    s = lax.axis_index("s")
    w = c * 16 + s
    base = c * HALF

    pltpu.sync_copy(uw, utab)
    _zero2d(acc, HROWS)

    def do_rows(nr, srcb=None, dstb=None):
        if srcb is None:
            srcb, dstb = srcb0, dstb0
        for r in range(nr):
            for g in range(8):
                sl = pl.ds(g * 16, 16)
                src16 = srcb[r, sl]
                dst16 = dstb[r, sl]
                word = lax.shift_right_logical(src16, 1)
                w16 = plsc.load_gather(
                    utab, [lax.shift_right_logical(word, 7),
                           lax.bitwise_and(word, 127)])
                sh = lax.shift_left(lax.bitwise_and(src16, 1), 4)
                v = plsc.bitcast(
                    lax.shift_left(lax.shift_right_logical(w16, sh), 16), _f32)
                local = dst16 - base
                m = lax.bitwise_and(local >= 0, local < HALF)
                idxc = jnp.where(m, local, 0)
                plsc.addupdate_scatter(
                    acc, [lax.shift_right_logical(idxc, 7),
                          lax.bitwise_and(idxc, 127)], v, mask=m)

    nch = A_RPT // 30

    def issue(k, sb, db, sem):
        row = s * A_RPT + k * 30
        pltpu.async_copy(ei.at[0, pl.ds(row, 30)], sb, sem)
        pltpu.async_copy(ei.at[1, pl.ds(row, 30)], db, sem)

    def drain(k, sb, db, sem):
        row = s * A_RPT + k * 30
        pltpu.make_async_copy(ei.at[0, pl.ds(row, 30)], sb, sem).wait()
        pltpu.make_async_copy(ei.at[1, pl.ds(row, 30)], db, sem).wait()

    issue(0, srcb0, dstb0, sem0)
    issue(1, srcb1, dstb1, sem1)

    def chunk(j, _):
        drain(2 * j, srcb0, dstb0, sem0)
        do_rows(30, srcb0, dstb0)

        @pl.when(j < nch // 2 - 1)
        def _():
            issue(2 * j + 2, srcb0, dstb0, sem0)
        drain(2 * j + 1, srcb1, dstb1, sem1)
        do_rows(30, srcb1, dstb1)

        @pl.when(j < nch // 2 - 1)
        def _():
            issue(2 * j + 3, srcb1, dstb1, sem1)
        return 0
    lax.fori_loop(0, nch // 2, chunk, 0)

    @pl.when(s < A_TAIL // 8)
    def _():
        row = 16 * A_RPT + s * 8
        pltpu.sync_copy(ei.at[0, pl.ds(row, 8)], srcb0.at[pl.ds(0, 8)])
        pltpu.sync_copy(ei.at[1, pl.ds(row, 8)], dstb0.at[pl.ds(0, 8)])
        do_rows(8)

    pltpu.sync_copy(acc, out.at[w])


def _prep_body(degp, x, dis_o, u1_o):
    deg = jnp.sum(degp[...], axis=0) + 1.0
    dis = lax.rsqrt(deg)
    dis_o[...] = dis
    u1_o[...] = dis * x[...]


def _gfull(g_ref):
    g = jnp.sum(g_ref[...], axis=1)
    return jnp.concatenate([g[0, :TROWS // 2], g[1, :TROWS // 2]], axis=0)


def _mid_body(g1, u1, dis, w1, b1, w2, u2_o):
    s1 = dis[...] * (_gfull(g1) + u1[...])
    acc = jnp.zeros_like(s1)
    for j in range(16):
        acc += jnp.maximum(s1 * w1[0, j] + b1[j], 0.0) * w2[j, 0]
    u2_o[...] = dis[...] * acc


def _fin_body(g2, u2, dis, b2, out_o):
    out_o[...] = dis[...] * (_gfull(g2) + u2[...]) + b2[0]


_vec2 = jax.ShapeDtypeStruct((TROWS, LANE), _f32)
_smem = pl.BlockSpec(memory_space=pltpu.SMEM)

_prep_call = pl.pallas_call(_prep_body, out_shape=(_vec2, _vec2))
_mid_call = pl.pallas_call(
    _mid_body,
    in_specs=[pl.BlockSpec(), pl.BlockSpec(), pl.BlockSpec(), _smem, _smem, _smem],
    out_shape=_vec2,
)
_fin_call = pl.pallas_call(
    _fin_body,
    in_specs=[pl.BlockSpec(), pl.BlockSpec(), pl.BlockSpec(), _smem],
    out_shape=_vec2,
)


def _pack_words(u2d):
    b16 = jax.lax.bitcast_convert_type(
        u2d.astype(jnp.bfloat16), jnp.uint16).astype(jnp.uint32)
    flat = b16.reshape(-1)
    words = flat[0::2] | (flat[1::2] << 16)
    return words.astype(_i32).reshape(UROWS, LANE)


def kernel(x, edge_index, W1, b1, W2, b2):
    n = x.shape[0]
    ei = edge_index.astype(_i32).reshape(2, ROWS, LANE)
    xp = jnp.pad(x[:, 0], (0, NPAD - n)).reshape(TROWS, LANE)

    degp = _deg_pass(ei)
    dis, u1 = _prep_call(degp, xp)
    g1 = _agg_pass(ei, _pack_words(u1)).reshape(2, 16, HROWS, LANE)
    u2 = _mid_call(g1, u1, dis, W1, b1, W2)
    g2 = _agg_pass(ei, _pack_words(u2)).reshape(2, 16, HROWS, LANE)
    out = _fin_call(g2, u2, dis, b2)
    return out.reshape(NPAD)[:n].reshape(n, 1)


# speedup vs baseline: 1.3302x; 1.0382x over previous
import functools

import jax
import jax.numpy as jnp
from jax import lax
from jax.experimental import pallas as pl
from jax.experimental.pallas import tpu as pltpu
from jax.experimental.pallas import tpu_sc as plsc

N_NODES = 100000
N_EDGES = 3200000

LANE = 128
ROWS = N_EDGES // LANE
NPAD = 100352
TROWS = NPAD // LANE
HALF = NPAD // 2
HROWS = 400
UROWS = NPAD // 256

D_RPT = 776
D_TAIL = ROWS - D_RPT * 32
A_RPT = 1560
A_TAIL = ROWS - A_RPT * 16

_mesh = plsc.VectorSubcoreMesh(core_axis_name="c", subcore_axis_name="s")
_cp = pltpu.CompilerParams(use_tc_tiling_on_sc=False, needs_layout_passes=False)

_f32 = jnp.float32
_i32 = jnp.int32


def _iota16():
    return lax.iota(_i32, 16)


def _zero2d(acc, nrows):
    def z(i, _):
        r16 = jnp.broadcast_to(lax.shift_right_logical(i, 3), (16,))
        c16 = lax.bitwise_and(i, 7) * 16 + _iota16()
        plsc.store_scatter(acc, [r16, c16], jnp.zeros((16,), _f32))
        return 0
    lax.fori_loop(0, nrows * 8, z, 0)


@functools.partial(
    pl.kernel,
    out_type=jax.ShapeDtypeStruct((32, TROWS, LANE), _f32),
    mesh=_mesh, compiler_params=_cp,
    scratch_types=[
        pltpu.VMEM((TROWS, LANE), _f32),
        pltpu.VMEM((24, LANE), _i32),
        pltpu.VMEM((24, LANE), _i32),
        pltpu.SemaphoreType.DMA,
        pltpu.SemaphoreType.DMA,
    ],
)
def _deg_pass(ei, out, acc, dstb0, dstb1, sem0, sem1):
    c = lax.axis_index("c")
    s = lax.axis_index("s")
    w = c * 16 + s
    _zero2d(acc, TROWS)

    ones = jnp.ones((16,), _f32)

    def do_rows(nr, dstb):
        for r in range(nr):
            for g in range(8):
                dst16 = dstb[r, pl.ds(g * 16, 16)]
                plsc.addupdate_scatter(
                    acc, [lax.shift_right_logical(dst16, 7),
                          lax.bitwise_and(dst16, 127)], ones)

    def issue(k, db, sem):
        pltpu.async_copy(ei.at[1, pl.ds(w * D_RPT + k * 24, 24)], db, sem)

    def drain(k, db, sem):
        pltpu.make_async_copy(
            ei.at[1, pl.ds(w * D_RPT + k * 24, 24)], db, sem).wait()

    issue(0, dstb0, sem0)
    issue(1, dstb1, sem1)

    def chunk(j, _):
        drain(2 * j, dstb0, sem0)
        do_rows(24, dstb0)

        @pl.when(j < 15)
        def _():
            issue(2 * j + 2, dstb0, sem0)
        drain(2 * j + 1, dstb1, sem1)
        do_rows(24, dstb1)

        @pl.when(j < 15)
        def _():
            issue(2 * j + 3, dstb1, sem1)
        return 0
    lax.fori_loop(0, 16, chunk, 0)
    pltpu.sync_copy(ei.at[1, pl.ds(w * D_RPT + 768, 8)], dstb0.at[pl.ds(0, 8)])
    do_rows(8, dstb0)

    @pl.when(w < D_TAIL // 8)
    def _():
        pltpu.sync_copy(ei.at[1, pl.ds(32 * D_RPT + w * 8, 8)],
                        dstb0.at[pl.ds(0, 8)])
        do_rows(8, dstb0)

    pltpu.sync_copy(acc, out.at[w])


@functools.partial(
    pl.kernel,
    out_type=jax.ShapeDtypeStruct((32, HROWS, LANE), _f32),
    mesh=_mesh, compiler_params=_cp,
    scratch_types=[
        pltpu.VMEM((UROWS, LANE), _i32),
        pltpu.VMEM((HROWS, LANE), _f32),
        pltpu.VMEM((30, LANE), _i32),
        pltpu.VMEM((30, LANE), _i32),
        pltpu.VMEM((30, LANE), _i32),
        pltpu.VMEM((30, LANE), _i32),
        pltpu.SemaphoreType.DMA,
        pltpu.SemaphoreType.DMA,
    ],
)
def _agg_pass(ei, uw, out, utab, acc, srcb0, dstb0, srcb1, dstb1, sem0, sem1):
    c = lax.axis_index("c")
    s = lax.axis_index("s")
    w = c * 16 + s
    base = c * HALF

    pltpu.sync_copy(uw, utab)
    _zero2d(acc, HROWS)

    def do_rows(nr, srcb=None, dstb=None):
        if srcb is None:
            srcb, dstb = srcb0, dstb0
        for r in range(nr):
            for g in range(8):
                sl = pl.ds(g * 16, 16)
                src16 = srcb[r, sl]
                dst16 = dstb[r, sl]
                word = lax.shift_right_logical(src16, 1)
                w16 = plsc.load_gather(
                    utab, [lax.shift_right_logical(word, 7),
                           lax.bitwise_and(word, 127)])
                sh = lax.shift_left(lax.bitwise_and(src16, 1), 4)
                v = plsc.bitcast(
                    lax.shift_left(lax.shift_right_logical(w16, sh), 16), _f32)
                local = dst16 - base
                m = lax.bitwise_and(local >= 0, local < HALF)
                idxc = jnp.where(m, local, 0)
                plsc.addupdate_scatter(
                    acc, [lax.shift_right_logical(idxc, 7),
                          lax.bitwise_and(idxc, 127)], v, mask=m)

    nch = A_RPT // 30

    def issue(k, sb, db, sem):
        row = s * A_RPT + k * 30
        pltpu.async_copy(ei.at[0, pl.ds(row, 30)], sb, sem)
        pltpu.async_copy(ei.at[1, pl.ds(row, 30)], db, sem)

    def drain(k, sb, db, sem):
        row = s * A_RPT + k * 30
        pltpu.make_async_copy(ei.at[0, pl.ds(row, 30)], sb, sem).wait()
        pltpu.make_async_copy(ei.at[1, pl.ds(row, 30)], db, sem).wait()

    issue(0, srcb0, dstb0, sem0)
    issue(1, srcb1, dstb1, sem1)

    def chunk(j, _):
        drain(2 * j, srcb0, dstb0, sem0)
        do_rows(30, srcb0, dstb0)

        @pl.when(j < nch // 2 - 1)
        def _():
            issue(2 * j + 2, srcb0, dstb0, sem0)
        drain(2 * j + 1, srcb1, dstb1, sem1)
        do_rows(30, srcb1, dstb1)

        @pl.when(j < nch // 2 - 1)
        def _():
            issue(2 * j + 3, srcb1, dstb1, sem1)
        return 0
    lax.fori_loop(0, nch // 2, chunk, 0)

    @pl.when(s < A_TAIL // 8)
    def _():
        row = 16 * A_RPT + s * 8
        pltpu.sync_copy(ei.at[0, pl.ds(row, 8)], srcb0.at[pl.ds(0, 8)])
        pltpu.sync_copy(ei.at[1, pl.ds(row, 8)], dstb0.at[pl.ds(0, 8)])
        do_rows(8)

    pltpu.sync_copy(acc, out.at[w])


def _prep_body(degp, x, dis_o, u1_o):
    deg = jnp.sum(degp[...], axis=0) + 1.0
    dis = lax.rsqrt(deg)
    dis_o[...] = dis
    u1_o[...] = dis * x[...]


def _gfull(g_ref):
    g = jnp.sum(g_ref[...], axis=1)
    return jnp.concatenate([g[0, :TROWS // 2], g[1, :TROWS // 2]], axis=0)


def _mid_body(g1, u1, dis, w1, b1, w2, u2_o):
    s1 = dis[...] * (_gfull(g1) + u1[...])
    acc = jnp.zeros_like(s1)
    for j in range(16):
        acc += jnp.maximum(s1 * w1[0, j] + b1[j], 0.0) * w2[j, 0]
    u2_o[...] = dis[...] * acc


def _fin_body(g2, u2, dis, b2, out_o):
    out_o[...] = dis[...] * (_gfull(g2) + u2[...]) + b2[0]


_vec2 = jax.ShapeDtypeStruct((TROWS, LANE), _f32)
_smem = pl.BlockSpec(memory_space=pltpu.SMEM)

_prep_call = pl.pallas_call(_prep_body, out_shape=(_vec2, _vec2))
_mid_call = pl.pallas_call(
    _mid_body,
    in_specs=[pl.BlockSpec(), pl.BlockSpec(), pl.BlockSpec(), _smem, _smem, _smem],
    out_shape=_vec2,
)
_fin_call = pl.pallas_call(
    _fin_body,
    in_specs=[pl.BlockSpec(), pl.BlockSpec(), pl.BlockSpec(), _smem],
    out_shape=_vec2,
)


def _pack_words(u2d):
    b16 = jax.lax.bitcast_convert_type(
        u2d.astype(jnp.bfloat16), jnp.uint16).astype(jnp.uint32)
    flat = b16.reshape(-1)
    words = flat[0::2] | (flat[1::2] << 16)
    return words.astype(_i32).reshape(UROWS, LANE)


def kernel(x, edge_index, W1, b1, W2, b2):
    n = x.shape[0]
    ei = edge_index.astype(_i32).reshape(2, ROWS, LANE)
    xp = jnp.pad(x[:, 0], (0, NPAD - n)).reshape(TROWS, LANE)

    degp = _deg_pass(ei)
    dis, u1 = _prep_call(degp, xp)
    g1 = _agg_pass(ei, _pack_words(u1)).reshape(2, 16, HROWS, LANE)
    u2 = _mid_call(g1, u1, dis, W1, b1, W2)
    g2 = _agg_pass(ei, _pack_words(u2)).reshape(2, 16, HROWS, LANE)
    out = _fin_call(g2, u2, dis, b2)
    return out.reshape(NPAD)[:n].reshape(n, 1)
